# baseline (speedup 1.0000x reference)
import jax
import jax.numpy as jnp
from jax.experimental import pallas as pl
from jax.experimental.pallas import tpu as pltpu

MAX_ATOMIC_NUM = 100
HIDDEN = 128
LATENT = 256
B = 100
N_PER = 100
N = B * N_PER
K = 15
CUTOFF = 10.0
N_RBF = 16
N_LAYERS = 2
G = 5

_HIGH = jax.lax.Precision.HIGHEST


def _dot(a, b):
    return jax.lax.dot_general(a, b, (((1,), (0,)), ((), ())),
                               precision=_HIGH,
                               preferred_element_type=jnp.float32)


def _silu(x):
    return x * jax.nn.sigmoid(x)


def _gemnet_kernel(frac_ref, types_ref, z_ref, lat_ref, invn_ref,
                   atom_emb_ref, Wz_ref, Ws_ref, Wd_ref, Wr_ref, Wu_ref,
                   wf_ref, Watom_ref, batom_ref,
                   Wch_ref, bch_ref, Wck_ref, bck_ref, Wcl_ref, bcl_ref,
                   coord_ref, alog_ref, lh_ref, lk_ref, ll_ref,
                   *scr):
    f32 = jnp.float32
    Ps = scr[0:G]
    us = scr[G:2 * G]
    PRs = scr[2 * G:3 * G]

    ioti = jax.lax.broadcasted_iota(jnp.int32, (N_PER, N_PER), 0).astype(f32)
    iotj = jax.lax.broadcasted_iota(jnp.int32, (N_PER, N_PER), 1).astype(f32)
    diag = jnp.where(ioti == iotj, f32(1e6), f32(0.0))

    carts, dist_nms, env_mats, dists = [], [], [], []
    for g in range(G):
        frac = frac_ref[g]
        lat = lat_ref[g]
        cart = (frac[:, 0:1] * lat[0:1, :]
                + frac[:, 1:2] * lat[1:2, :]
                + frac[:, 2:3] * lat[2:3, :])
        dist2 = jnp.zeros((N_PER, N_PER), f32)
        for e in range(3):
            col = cart[:, e:e + 1]
            row = col.reshape(1, N_PER)
            dd = col - row
            dist2 = dist2 + dd * dd
        dist_nm = jnp.sqrt(dist2 + 1e-12)
        env_mat = 0.5 * (jnp.cos(jnp.pi * jnp.clip(dist_nm / CUTOFF,
                                                   0.0, 1.0)) + 1.0)
        carts.append(cart)
        dist_nms.append(dist_nm)
        env_mats.append(env_mat)
        dists.append(dist_nm + diag)

    def topk_body(k, Ds):
        out = []
        for g in range(G):
            D = Ds[g]
            rowmin = jnp.min(D, axis=1, keepdims=True)
            ismin = D == rowmin
            idx = jnp.min(jnp.where(ismin, iotj, f32(N_PER)), axis=1,
                          keepdims=True)
            sel = iotj == idx
            Ps[g][k] = sel.astype(f32)
            out.append(jnp.where(sel, f32(1e6), D))
        return tuple(out)

    jax.lax.fori_loop(0, K, topk_body, tuple(dists), unroll=False)

    gamma = CUTOFF / N_RBF
    centers = (jax.lax.broadcasted_iota(jnp.int32, (1, N_RBF), 1).astype(f32)
               * f32(CUTOFF / (N_RBF - 1)))
    for k in range(K):
        for g in range(G):
            Pk = Ps[g][k]
            d = jnp.sum(Pk * dist_nms[g], axis=1, keepdims=True)
            e = jnp.sum(Pk * env_mats[g], axis=1, keepdims=True)
            gcart = _dot(Pk, carts[g])
            vec = gcart - carts[g]
            us[g][k] = vec / d
            rbf = jnp.exp(-((d - centers) ** 2) / f32(2.0 * gamma * gamma))
            PRs[g][k] = jnp.concatenate([Pk, rbf * e], axis=1)

    tiot = jax.lax.broadcasted_iota(
        jnp.int32, (N_PER, MAX_ATOMIC_NUM), 1).astype(f32)
    hcur = []
    for g in range(G):
        type_oh = (types_ref[g] == tiot).astype(f32)
        hcur.append(_dot(type_oh, atom_emb_ref[...])
                    + _dot(z_ref[g], Wz_ref[...]))

    for l in range(N_LAYERS):
        Wr_l = Wr_ref[l]
        hds, cats, aggs = [], [], []
        for g in range(G):
            hs = _dot(hcur[g], Ws_ref[l])
            hds.append(_dot(hcur[g], Wd_ref[l]))
            cats.append(jnp.concatenate([hs, Wr_l], axis=0))
            aggs.append(jnp.zeros((N_PER, HIDDEN), f32))
        for k in range(K):
            for g in range(G):
                ghs_rw = _dot(PRs[g][k], cats[g])
                aggs[g] = aggs[g] + _silu(ghs_rw + hds[g])
        for g in range(G):
            hcur[g] = hcur[g] + _silu(_dot(aggs[g], Wu_ref[l]))

    avs, coords = [], []
    for g in range(G):
        avs.append(_dot(hcur[g], wf_ref[...]))
        coords.append(jnp.zeros((N_PER, 3), f32))
    arows = [avs[g].reshape(1, N_PER) for g in range(G)]
    for k in range(K):
        for g in range(G):
            s = (jnp.sum(Ps[g][k] * arows[g], axis=1, keepdims=True)
                 + avs[g])
            coords[g] = coords[g] + s * us[g][k]
    for g in range(G):
        coord_ref[g] = coords[g]

    for g in range(G):
        alog_ref[g] = _dot(hcur[g], Watom_ref[...]) + batom_ref[...]
        repr_ = (jnp.sum(hcur[g], axis=0, keepdims=True)
                 * invn_ref[g])
        lh_ref[g] = _dot(repr_, Wch_ref[...]) + bch_ref[...]
        lk_ref[g] = _dot(repr_, Wck_ref[...]) + bck_ref[...]
        ll_ref[g] = _dot(repr_, Wcl_ref[...]) + bcl_ref[...]


def _lattice(lengths, angles):
    a, b, c = lengths[:, 0], lengths[:, 1], lengths[:, 2]
    al, be, ga = [jnp.deg2rad(angles[:, i]) for i in range(3)]
    cos_al, cos_be, cos_ga = jnp.cos(al), jnp.cos(be), jnp.cos(ga)
    sin_ga = jnp.clip(jnp.sin(ga), 1e-6)
    zeros = jnp.zeros_like(a)
    vx = jnp.stack([a, zeros, zeros], -1)
    vy = jnp.stack([b * cos_ga, b * sin_ga, zeros], -1)
    cx = c * cos_be
    cy = c * (cos_al - cos_be * cos_ga) / sin_ga
    cz = jnp.sqrt(jnp.clip(c ** 2 - cx ** 2 - cy ** 2, 1e-8))
    vz = jnp.stack([cx, cy, cz], -1)
    return jnp.stack([vx, vy, vz], 1)


@jax.jit
def kernel(z, pred_frac_coords, pred_atom_types, num_atoms, lengths, angles,
           atom_emb, Wz, Ws, Wd, Wr, Wu, w_f, W_atom, b_atom,
           W_ch, b_ch, W_ck, b_ck, W_cl, b_cl):
    f32 = jnp.float32
    lat = _lattice(lengths, angles)

    frac = pred_frac_coords.reshape(B, N_PER, 3)
    types = pred_atom_types.astype(f32).reshape(B, N_PER, 1)
    zz = z.reshape(B, 1, LATENT)
    invn = (1.0 / num_atoms.astype(f32)).reshape(B, 1, 1)

    grid = (B // G,)

    def gmap(i):
        return (i, 0, 0)

    def wmap2(i):
        return (0, 0)

    def wmap3(i):
        return (0, 0, 0)

    in_specs = [
        pl.BlockSpec((G, N_PER, 3), gmap),
        pl.BlockSpec((G, N_PER, 1), gmap),
        pl.BlockSpec((G, 1, LATENT), gmap),
        pl.BlockSpec((G, 3, 3), gmap),
        pl.BlockSpec((G, 1, 1), gmap),
        pl.BlockSpec((MAX_ATOMIC_NUM, HIDDEN), wmap2),
        pl.BlockSpec((LATENT, HIDDEN), wmap2),
        pl.BlockSpec((N_LAYERS, HIDDEN, HIDDEN), wmap3),
        pl.BlockSpec((N_LAYERS, HIDDEN, HIDDEN), wmap3),
        pl.BlockSpec((N_LAYERS, N_RBF, HIDDEN), wmap3),
        pl.BlockSpec((N_LAYERS, HIDDEN, HIDDEN), wmap3),
        pl.BlockSpec((HIDDEN, 1), wmap2),
        pl.BlockSpec((HIDDEN, MAX_ATOMIC_NUM), wmap2),
        pl.BlockSpec((1, MAX_ATOMIC_NUM), wmap2),
        pl.BlockSpec((HIDDEN, 5), wmap2),
        pl.BlockSpec((1, 5), wmap2),
        pl.BlockSpec((HIDDEN, 5), wmap2),
        pl.BlockSpec((1, 5), wmap2),
        pl.BlockSpec((HIDDEN, 5), wmap2),
        pl.BlockSpec((1, 5), wmap2),
    ]
    out_specs = [
        pl.BlockSpec((G, N_PER, 3), gmap),
        pl.BlockSpec((G, N_PER, MAX_ATOMIC_NUM), gmap),
        pl.BlockSpec((G, 1, 5), gmap),
        pl.BlockSpec((G, 1, 5), gmap),
        pl.BlockSpec((G, 1, 5), gmap),
    ]
    out_shape = [
        jax.ShapeDtypeStruct((B, N_PER, 3), f32),
        jax.ShapeDtypeStruct((B, N_PER, MAX_ATOMIC_NUM), f32),
        jax.ShapeDtypeStruct((B, 1, 5), f32),
        jax.ShapeDtypeStruct((B, 1, 5), f32),
        jax.ShapeDtypeStruct((B, 1, 5), f32),
    ]
    scratch_shapes = (
        [pltpu.VMEM((K, N_PER, N_PER), f32) for _ in range(G)]
        + [pltpu.VMEM((K, N_PER, 3), f32) for _ in range(G)]
        + [pltpu.VMEM((K, N_PER, N_PER + N_RBF), f32)
           for _ in range(G)])

    coord, alog, lh, lk, ll = pl.pallas_call(
        _gemnet_kernel,
        grid=grid,
        in_specs=in_specs,
        out_specs=out_specs,
        out_shape=out_shape,
        scratch_shapes=scratch_shapes,
    )(frac, types, zz, lat, invn,
      atom_emb, Wz, Ws, Wd, Wr, Wu,
      w_f.reshape(HIDDEN, 1), W_atom, b_atom.reshape(1, MAX_ATOMIC_NUM),
      W_ch, b_ch.reshape(1, 5), W_ck, b_ck.reshape(1, 5),
      W_cl, b_cl.reshape(1, 5))

    return (coord.reshape(N, 3), alog.reshape(N, MAX_ATOMIC_NUM),
            (lh.reshape(B, 5), lk.reshape(B, 5), ll.reshape(B, 5)))


# speedup vs baseline: 1.0884x; 1.0884x over previous
import jax
import jax.numpy as jnp
from jax.experimental import pallas as pl
from jax.experimental.pallas import tpu as pltpu

MAX_ATOMIC_NUM = 100
HIDDEN = 128
LATENT = 256
B = 100
N_PER = 100
N = B * N_PER
K = 15
CUTOFF = 10.0
N_RBF = 16
N_LAYERS = 2
G = 4

_HIGH = jax.lax.Precision.HIGHEST


def _dot(a, b):
    return jax.lax.dot_general(a, b, (((1,), (0,)), ((), ())),
                               precision=_HIGH,
                               preferred_element_type=jnp.float32)


def _silu(x):
    return x * jax.nn.sigmoid(x)


def _gemnet_kernel(frac_ref, types_ref, z_ref, lat_ref, invn_ref,
                   atom_emb_ref, Wz_ref, Ws_ref, Wd_ref, Wr_ref, Wu_ref,
                   wf_ref, Watom_ref, batom_ref,
                   Wch_ref, bch_ref, Wck_ref, bck_ref, Wcl_ref, bcl_ref,
                   coord_ref, alog_ref, lh_ref, lk_ref, ll_ref,
                   *scr):
    f32 = jnp.float32
    Ps = scr[0:G]
    us = scr[G:2 * G]
    PRs = scr[2 * G:3 * G]

    ioti = jax.lax.broadcasted_iota(jnp.int32, (N_PER, N_PER), 0).astype(f32)
    iotj = jax.lax.broadcasted_iota(jnp.int32, (N_PER, N_PER), 1).astype(f32)
    diag = jnp.where(ioti == iotj, f32(1e6), f32(0.0))

    carts, dist_nms, env_mats, dists, crows = [], [], [], [], []
    for g in range(G):
        frac = frac_ref[g]
        lat = lat_ref[g]
        cart = (frac[:, 0:1] * lat[0:1, :]
                + frac[:, 1:2] * lat[1:2, :]
                + frac[:, 2:3] * lat[2:3, :])
        dist2 = jnp.zeros((N_PER, N_PER), f32)
        rows = []
        for e in range(3):
            col = cart[:, e:e + 1]
            row = col.reshape(1, N_PER)
            rows.append(row)
            dd = col - row
            dist2 = dist2 + dd * dd
        crows.append(rows)
        dist_nm = jnp.sqrt(dist2 + 1e-12)
        env_mat = 0.5 * (jnp.cos(jnp.pi * jnp.clip(dist_nm / CUTOFF,
                                                   0.0, 1.0)) + 1.0)
        carts.append(cart)
        dist_nms.append(dist_nm)
        env_mats.append(env_mat)
        dists.append(dist_nm + diag)

    def topk_body(k, Ds):
        out = []
        for g in range(G):
            D = Ds[g]
            rowmin = jnp.min(D, axis=1, keepdims=True)
            ismin = D == rowmin
            idx = jnp.min(jnp.where(ismin, iotj, f32(N_PER)), axis=1,
                          keepdims=True)
            sel = iotj == idx
            Ps[g][k] = sel.astype(f32)
            out.append(jnp.where(sel, f32(1e6), D))
        return tuple(out)

    jax.lax.fori_loop(0, K, topk_body, tuple(dists), unroll=False)

    gamma = CUTOFF / N_RBF
    centers = (jax.lax.broadcasted_iota(jnp.int32, (1, N_RBF), 1).astype(f32)
               * f32(CUTOFF / (N_RBF - 1)))
    for k in range(K):
        for g in range(G):
            Pk = Ps[g][k]
            d = jnp.sum(Pk * dist_nms[g], axis=1, keepdims=True)
            e = jnp.sum(Pk * env_mats[g], axis=1, keepdims=True)
            gcart = jnp.concatenate(
                [jnp.sum(Pk * crows[g][0], axis=1, keepdims=True),
                 jnp.sum(Pk * crows[g][1], axis=1, keepdims=True),
                 jnp.sum(Pk * crows[g][2], axis=1, keepdims=True)], axis=1)
            vec = gcart - carts[g]
            us[g][k] = vec / d
            rbf = jnp.exp(-((d - centers) ** 2) / f32(2.0 * gamma * gamma))
            PRs[g][k] = jnp.concatenate([Pk, rbf * e], axis=1)

    tiot = jax.lax.broadcasted_iota(
        jnp.int32, (N_PER, MAX_ATOMIC_NUM), 1).astype(f32)
    hcur = []
    for g in range(G):
        type_oh = (types_ref[g] == tiot).astype(f32)
        hcur.append(_dot(type_oh, atom_emb_ref[...])
                    + _dot(z_ref[g], Wz_ref[...]))

    for l in range(N_LAYERS):
        Wr_l = Wr_ref[l]
        hds, cats, aggs = [], [], []
        for g in range(G):
            hs = _dot(hcur[g], Ws_ref[l])
            hds.append(_dot(hcur[g], Wd_ref[l]))
            cats.append(jnp.concatenate([hs, Wr_l], axis=0))
            aggs.append(jnp.zeros((N_PER, HIDDEN), f32))
        for k in range(K):
            for g in range(G):
                ghs_rw = _dot(PRs[g][k], cats[g])
                aggs[g] = aggs[g] + _silu(ghs_rw + hds[g])
        for g in range(G):
            hcur[g] = hcur[g] + _silu(_dot(aggs[g], Wu_ref[l]))

    avs, coords = [], []
    for g in range(G):
        avs.append(_dot(hcur[g], wf_ref[...]))
        coords.append(jnp.zeros((N_PER, 3), f32))
    arows = [avs[g].reshape(1, N_PER) for g in range(G)]
    for k in range(K):
        for g in range(G):
            s = (jnp.sum(Ps[g][k] * arows[g], axis=1, keepdims=True)
                 + avs[g])
            coords[g] = coords[g] + s * us[g][k]
    for g in range(G):
        coord_ref[g] = coords[g]

    for g in range(G):
        alog_ref[g] = _dot(hcur[g], Watom_ref[...]) + batom_ref[...]
        repr_ = (jnp.sum(hcur[g], axis=0, keepdims=True)
                 * invn_ref[g])
        lh_ref[g] = _dot(repr_, Wch_ref[...]) + bch_ref[...]
        lk_ref[g] = _dot(repr_, Wck_ref[...]) + bck_ref[...]
        ll_ref[g] = _dot(repr_, Wcl_ref[...]) + bcl_ref[...]


def _lattice(lengths, angles):
    a, b, c = lengths[:, 0], lengths[:, 1], lengths[:, 2]
    al, be, ga = [jnp.deg2rad(angles[:, i]) for i in range(3)]
    cos_al, cos_be, cos_ga = jnp.cos(al), jnp.cos(be), jnp.cos(ga)
    sin_ga = jnp.clip(jnp.sin(ga), 1e-6)
    zeros = jnp.zeros_like(a)
    vx = jnp.stack([a, zeros, zeros], -1)
    vy = jnp.stack([b * cos_ga, b * sin_ga, zeros], -1)
    cx = c * cos_be
    cy = c * (cos_al - cos_be * cos_ga) / sin_ga
    cz = jnp.sqrt(jnp.clip(c ** 2 - cx ** 2 - cy ** 2, 1e-8))
    vz = jnp.stack([cx, cy, cz], -1)
    return jnp.stack([vx, vy, vz], 1)


@jax.jit
def kernel(z, pred_frac_coords, pred_atom_types, num_atoms, lengths, angles,
           atom_emb, Wz, Ws, Wd, Wr, Wu, w_f, W_atom, b_atom,
           W_ch, b_ch, W_ck, b_ck, W_cl, b_cl):
    f32 = jnp.float32
    lat = _lattice(lengths, angles)

    frac = pred_frac_coords.reshape(B, N_PER, 3)
    types = pred_atom_types.astype(f32).reshape(B, N_PER, 1)
    zz = z.reshape(B, 1, LATENT)
    invn = (1.0 / num_atoms.astype(f32)).reshape(B, 1, 1)

    grid = (B // G,)

    def gmap(i):
        return (i, 0, 0)

    def wmap2(i):
        return (0, 0)

    def wmap3(i):
        return (0, 0, 0)

    in_specs = [
        pl.BlockSpec((G, N_PER, 3), gmap),
        pl.BlockSpec((G, N_PER, 1), gmap),
        pl.BlockSpec((G, 1, LATENT), gmap),
        pl.BlockSpec((G, 3, 3), gmap),
        pl.BlockSpec((G, 1, 1), gmap),
        pl.BlockSpec((MAX_ATOMIC_NUM, HIDDEN), wmap2),
        pl.BlockSpec((LATENT, HIDDEN), wmap2),
        pl.BlockSpec((N_LAYERS, HIDDEN, HIDDEN), wmap3),
        pl.BlockSpec((N_LAYERS, HIDDEN, HIDDEN), wmap3),
        pl.BlockSpec((N_LAYERS, N_RBF, HIDDEN), wmap3),
        pl.BlockSpec((N_LAYERS, HIDDEN, HIDDEN), wmap3),
        pl.BlockSpec((HIDDEN, 1), wmap2),
        pl.BlockSpec((HIDDEN, MAX_ATOMIC_NUM), wmap2),
        pl.BlockSpec((1, MAX_ATOMIC_NUM), wmap2),
        pl.BlockSpec((HIDDEN, 5), wmap2),
        pl.BlockSpec((1, 5), wmap2),
        pl.BlockSpec((HIDDEN, 5), wmap2),
        pl.BlockSpec((1, 5), wmap2),
        pl.BlockSpec((HIDDEN, 5), wmap2),
        pl.BlockSpec((1, 5), wmap2),
    ]
    out_specs = [
        pl.BlockSpec((G, N_PER, 3), gmap),
        pl.BlockSpec((G, N_PER, MAX_ATOMIC_NUM), gmap),
        pl.BlockSpec((G, 1, 5), gmap),
        pl.BlockSpec((G, 1, 5), gmap),
        pl.BlockSpec((G, 1, 5), gmap),
    ]
    out_shape = [
        jax.ShapeDtypeStruct((B, N_PER, 3), f32),
        jax.ShapeDtypeStruct((B, N_PER, MAX_ATOMIC_NUM), f32),
        jax.ShapeDtypeStruct((B, 1, 5), f32),
        jax.ShapeDtypeStruct((B, 1, 5), f32),
        jax.ShapeDtypeStruct((B, 1, 5), f32),
    ]
    scratch_shapes = (
        [pltpu.VMEM((K, N_PER, N_PER), f32) for _ in range(G)]
        + [pltpu.VMEM((K, N_PER, 3), f32) for _ in range(G)]
        + [pltpu.VMEM((K, N_PER, N_PER + N_RBF), f32)
           for _ in range(G)])

    coord, alog, lh, lk, ll = pl.pallas_call(
        _gemnet_kernel,
        grid=grid,
        in_specs=in_specs,
        out_specs=out_specs,
        out_shape=out_shape,
        scratch_shapes=scratch_shapes,
    )(frac, types, zz, lat, invn,
      atom_emb, Wz, Ws, Wd, Wr, Wu,
      w_f.reshape(HIDDEN, 1), W_atom, b_atom.reshape(1, MAX_ATOMIC_NUM),
      W_ch, b_ch.reshape(1, 5), W_ck, b_ck.reshape(1, 5),
      W_cl, b_cl.reshape(1, 5))

    return (coord.reshape(N, 3), alog.reshape(N, MAX_ATOMIC_NUM),
            (lh.reshape(B, 5), lk.reshape(B, 5), ll.reshape(B, 5)))
